# baseline (speedup 1.0000x reference)
import jax
import jax.numpy as jnp
import numpy as np
from jax import lax
from jax.experimental import pallas as pl
from jax.experimental.pallas import tpu as pltpu

N = 4096
H = 256
W = 256
C = 3
G = 64
ROWS = 8
COLS = 128
NHALF = W // COLS
NE = N * NHALF
NCH = NE // G
PX = ROWS * COLS
RCUT = 8.5


def _raster_kernel(bounds_ref, cx_ref, cy_ref, a_ref, b_ref, c_ref, vt_ref,
                   out_ref):
    i = pl.program_id(0)
    h = pl.program_id(1)
    elo = bounds_ref[0, i, h]
    ehi = bounds_ref[1, i, h]
    lo = elo // G
    hi = (ehi + G - 1) // G
    pix = lax.broadcasted_iota(jnp.int32, (G, PX), 1)
    gx = (pix & (COLS - 1)).astype(jnp.float32) + (h * COLS).astype(jnp.float32)
    gy = (pix >> 7).astype(jnp.float32) + (i * ROWS).astype(jnp.float32)

    def chunk(j, acc):
        jm = jnp.minimum(j, NCH - 1)
        cxc = cx_ref[jm].reshape(G, 1)
        cyc = cy_ref[jm].reshape(G, 1)
        ac = a_ref[jm].reshape(G, 1)
        bc = b_ref[jm].reshape(G, 1)
        cc = c_ref[jm].reshape(G, 1)
        dx = gx - cxc
        dy = gy - cyc
        power = (ac * dx) * dx + ((bc * dx) + (cc * dy)) * dy
        ent = lax.broadcasted_iota(jnp.int32, (G, 1), 0) + j * G
        live = ((ent >= elo) & (ent < ehi)).astype(jnp.float32)
        w = jnp.exp(power) * live
        return acc + jnp.dot(vt_ref[:, jm], w, preferred_element_type=jnp.float32)

    def body(k, acc):
        j = lo + 2 * k
        return chunk(j + 1, chunk(j, acc))

    npair = (hi - lo + 1) // 2
    acc = lax.fori_loop(0, npair, body, jnp.zeros((C, PX), jnp.float32))
    out_ref[...] = acc.reshape(C, ROWS, COLS)


def kernel(xy, scaling, rotation, values, opacity):
    xy_t = jnp.tanh(xy)
    s = jnp.abs(scaling) + 0.3
    theta = jax.nn.sigmoid(rotation[:, 0]) * 2.0 * np.pi
    cos_t = jnp.cos(theta)
    sin_t = jnp.sin(theta)
    s0 = s[:, 0]
    s1 = s[:, 1]
    a = cos_t * cos_t * s0 * s0 + sin_t * sin_t * s1 * s1
    b = cos_t * sin_t * (s0 * s0 - s1 * s1)
    c = sin_t * sin_t * s0 * s0 + cos_t * cos_t * s1 * s1
    det = a * c - b * b
    cx = 0.5 * W * (xy_t[:, 0] + 1.0) - 0.5
    cy = 0.5 * H * (xy_t[:, 1] + 1.0) - 0.5
    vop = values * opacity

    HOFF = 1e4
    FAR = 1e9
    ghalf = jnp.tile(cx, NHALF).reshape(NHALF, N)
    hbase = (jnp.arange(NHALF, dtype=jnp.float32) * COLS)[:, None]
    relevant = (ghalf >= hbase - RCUT) & (ghalf <= hbase + (COLS - 1) + RCUT)
    key = (jnp.tile(cy, NHALF).reshape(NHALF, N) + jnp.float32(HOFF) * hbase / COLS
           + jnp.where(relevant, 0.0, jnp.float32(FAR))).reshape(NE)
    key_s, eidx = lax.sort((key, lax.iota(jnp.int32, NE)), num_keys=1)
    gidx = eidx & (N - 1)

    params8 = jnp.stack(
        [cx, cy, -0.5 * c / det, b / det, -0.5 * a / det,
         vop[:, 0], vop[:, 1], vop[:, 2]], axis=1)
    sorted8 = params8[gidx]

    grid = H // ROWS
    rowlo = jnp.arange(grid, dtype=jnp.float32) * ROWS
    qlo = (rowlo[:, None] - RCUT) + HOFF * jnp.arange(NHALF)[None, :]
    qhi = (rowlo[:, None] + (ROWS - 1) + RCUT) + HOFF * jnp.arange(NHALF)[None, :]
    starts = jnp.searchsorted(key_s, qlo.reshape(-1))
    ends = jnp.searchsorted(key_s, qhi.reshape(-1), side='right')
    bounds = jnp.stack([starts, ends]).astype(jnp.int32).reshape(2, grid, NHALF)

    q = lambda k: sorted8[:, k].reshape(NCH, G)
    vt = sorted8[:, 5:8].T.reshape(C, NCH, G)

    full = lambda shp: pl.BlockSpec(shp, lambda *_: tuple(0 for _ in shp))
    out = pl.pallas_call(
        _raster_kernel,
        grid_spec=pltpu.PrefetchScalarGridSpec(
            num_scalar_prefetch=1,
            grid=(grid, NHALF),
            in_specs=[full((NCH, G))] * 5 + [full((C, NCH, G))],
            out_specs=pl.BlockSpec((C, ROWS, COLS), lambda i, h, b: (0, i, h)),
        ),
        out_shape=jax.ShapeDtypeStruct((C, H, W), jnp.float32),
    )(bounds, q(0), q(1), q(2), q(3), q(4), vt)

    return out.reshape(1, C, H, W)


# speedup vs baseline: 1.5258x; 1.5258x over previous
import jax
import jax.numpy as jnp
import numpy as np
from jax import lax
from jax.experimental import pallas as pl
from jax.experimental.pallas import tpu as pltpu

N = 4096
H = 256
W = 256
C = 3
G = 64
ROWS = 8
NCH = N // G
PX = ROWS * W
RCUT = 8.5


def _raster_kernel(bounds_ref, cx_ref, cy_ref, a_ref, b_ref, c_ref, vt_ref,
                   out_ref):
    i = pl.program_id(0)
    elo = bounds_ref[0, i]
    ehi = bounds_ref[1, i]
    lo = elo // G
    hi = (ehi + G - 1) // G
    pix = lax.broadcasted_iota(jnp.int32, (G, PX), 1)
    gx = (pix & (W - 1)).astype(jnp.float32)
    gy = (pix >> 8).astype(jnp.float32) + (i * ROWS).astype(jnp.float32)

    def chunk(j, acc):
        jm = jnp.minimum(j, NCH - 1)
        cxc = cx_ref[jm].reshape(G, 1)
        cyc = cy_ref[jm].reshape(G, 1)
        ac = a_ref[jm].reshape(G, 1)
        bc = b_ref[jm].reshape(G, 1)
        cc = c_ref[jm].reshape(G, 1)
        dx = gx - cxc
        dy = gy - cyc
        power = (ac * dx) * dx + ((bc * dx) + (cc * dy)) * dy
        ent = lax.broadcasted_iota(jnp.int32, (G, 1), 0) + j * G
        live = (ent < ehi).astype(jnp.float32)
        w = jnp.exp(power) * live
        return acc + jnp.dot(vt_ref[:, jm], w, preferred_element_type=jnp.float32)

    def body(k, acc):
        j = lo + 2 * k
        return chunk(j + 1, chunk(j, acc))

    npair = (hi - lo + 1) // 2
    acc = lax.fori_loop(0, npair, body, jnp.zeros((C, PX), jnp.float32))
    out_ref[...] = acc.reshape(C, ROWS, W)


def kernel(xy, scaling, rotation, values, opacity):
    xy_t = jnp.tanh(xy)
    s = jnp.abs(scaling) + 0.3
    theta = jax.nn.sigmoid(rotation[:, 0]) * 2.0 * np.pi
    cos_t = jnp.cos(theta)
    sin_t = jnp.sin(theta)
    s0 = s[:, 0]
    s1 = s[:, 1]
    a = cos_t * cos_t * s0 * s0 + sin_t * sin_t * s1 * s1
    b = cos_t * sin_t * (s0 * s0 - s1 * s1)
    c = sin_t * sin_t * s0 * s0 + cos_t * cos_t * s1 * s1
    det = a * c - b * b
    cx = 0.5 * W * (xy_t[:, 0] + 1.0) - 0.5
    cy = 0.5 * H * (xy_t[:, 1] + 1.0) - 0.5
    vop = values * opacity

    cy_s, cx_s, a_s, b_s, c_s, v0, v1, v2 = lax.sort(
        (cy, cx, -0.5 * c / det, b / det, -0.5 * a / det,
         vop[:, 0], vop[:, 1], vop[:, 2]), num_keys=1)

    grid = H // ROWS
    rowlo = jnp.arange(grid, dtype=jnp.float32) * ROWS
    starts = jnp.searchsorted(cy_s, rowlo - RCUT)
    ends = jnp.searchsorted(cy_s, rowlo + (ROWS - 1) + RCUT, side='right')
    bounds = jnp.stack([starts, ends]).astype(jnp.int32)

    q = lambda x: x.reshape(NCH, G)
    vt = jnp.stack([v0, v1, v2]).reshape(C, NCH, G)

    full = lambda shp: pl.BlockSpec(shp, lambda *_: tuple(0 for _ in shp))
    out = pl.pallas_call(
        _raster_kernel,
        grid_spec=pltpu.PrefetchScalarGridSpec(
            num_scalar_prefetch=1,
            grid=(grid,),
            in_specs=[full((NCH, G))] * 5 + [full((C, NCH, G))],
            out_specs=pl.BlockSpec((C, ROWS, W), lambda i, b: (0, i, 0)),
        ),
        out_shape=jax.ShapeDtypeStruct((C, H, W), jnp.float32),
    )(bounds, q(cx_s), q(cy_s), q(a_s), q(b_s), q(c_s), vt)

    return out.reshape(1, C, H, W)


# speedup vs baseline: 1.6473x; 1.0796x over previous
import jax
import jax.numpy as jnp
import numpy as np
from jax import lax
from jax.experimental import pallas as pl
from jax.experimental.pallas import tpu as pltpu

N = 4096
H = 256
W = 256
C = 3
G = 64
ROWS = 8
NCH = N // G
PX = ROWS * W
RCUT = 8.5


def _raster_kernel(bounds_ref, cx_ref, cy_ref, a_ref, b_ref, c_ref, vt_ref,
                   out_ref):
    i = pl.program_id(0)
    elo = bounds_ref[0, i]
    ehi = bounds_ref[1, i]
    lo = elo // G
    hi = (ehi + G - 1) // G
    pix = lax.broadcasted_iota(jnp.int32, (G, PX), 1)
    gx = (pix & (W - 1)).astype(jnp.float32)
    gy = (pix >> 8).astype(jnp.float32) + (i * ROWS).astype(jnp.float32)

    def chunk(j, acc):
        jm = jnp.minimum(j, NCH - 1)
        cxc = cx_ref[jm].reshape(G, 1)
        cyc = cy_ref[jm].reshape(G, 1)
        ac = a_ref[jm].reshape(G, 1)
        bc = b_ref[jm].reshape(G, 1)
        cc = c_ref[jm].reshape(G, 1)
        dx = gx - cxc
        dy = gy - cyc
        power = (ac * dx) * dx + ((bc * dx) + (cc * dy)) * dy
        ent = lax.broadcasted_iota(jnp.int32, (G, 1), 0) + j * G
        live = (ent < ehi).astype(jnp.float32)
        w = jnp.exp(power) * live
        return acc + jnp.dot(vt_ref[:, jm], w, preferred_element_type=jnp.float32)

    def body(k, acc):
        j = lo + 2 * k
        return chunk(j + 1, chunk(j, acc))

    npair = (hi - lo + 1) // 2
    acc = lax.fori_loop(0, npair, body, jnp.zeros((C, PX), jnp.float32))
    out_ref[...] = acc.reshape(C, ROWS, W)


def _project_kernel(p_ref, cy_ref, cx_ref, a_ref, b_ref, c_ref,
                    v0_ref, v1_ref, v2_ref, bounds_ref):
    xt = jnp.tanh(p_ref[0:1, :])
    yt = jnp.tanh(p_ref[1:2, :])
    s0 = jnp.abs(p_ref[2:3, :]) + 0.3
    s1 = jnp.abs(p_ref[3:4, :]) + 0.3
    theta = jax.nn.sigmoid(p_ref[4:5, :]) * (2.0 * np.pi)
    cos_t = jnp.cos(theta)
    sin_t = jnp.sin(theta)
    a = cos_t * cos_t * s0 * s0 + sin_t * sin_t * s1 * s1
    b = cos_t * sin_t * (s0 * s0 - s1 * s1)
    c = sin_t * sin_t * s0 * s0 + cos_t * cos_t * s1 * s1
    inv_det = 1.0 / (a * c - b * b)
    cx = 0.5 * W * (xt + 1.0) - 0.5
    cy = 0.5 * H * (yt + 1.0) - 0.5
    op = p_ref[8:9, :]
    cy_ref[...] = cy.reshape(N)
    cx_ref[...] = cx.reshape(N)
    a_ref[...] = (-0.5 * c * inv_det).reshape(N)
    b_ref[...] = (b * inv_det).reshape(N)
    c_ref[...] = (-0.5 * a * inv_det).reshape(N)
    v0_ref[...] = (p_ref[5:6, :] * op).reshape(N)
    v1_ref[...] = (p_ref[6:7, :] * op).reshape(N)
    v2_ref[...] = (p_ref[7:8, :] * op).reshape(N)
    grid = H // ROWS
    rowlo = lax.broadcasted_iota(jnp.int32, (grid, 1), 0).astype(jnp.float32) * ROWS
    lo_cnt = jnp.sum((cy < rowlo - RCUT).astype(jnp.int32), axis=1)
    hi_cnt = jnp.sum((cy <= rowlo + (ROWS - 1) + RCUT).astype(jnp.int32), axis=1)
    bounds_ref[0:1, :] = lo_cnt.reshape(1, grid)
    bounds_ref[1:2, :] = hi_cnt.reshape(1, grid)


def kernel(xy, scaling, rotation, values, opacity):
    grid = H // ROWS
    packed = jnp.concatenate(
        [xy, scaling, rotation, values, opacity], axis=1).T
    o1 = jax.ShapeDtypeStruct((N,), jnp.float32)
    cy, cx, aq, bq, cq, v0, v1, v2, bounds = pl.pallas_call(
        _project_kernel,
        out_shape=[o1] * 8 + [jax.ShapeDtypeStruct((2, grid), jnp.int32)],
    )(packed)

    cy_s, cx_s, a_s, b_s, c_s, v0, v1, v2 = lax.sort(
        (cy, cx, aq, bq, cq, v0, v1, v2), num_keys=1)

    q = lambda x: x.reshape(NCH, G)
    vt = jnp.stack([v0, v1, v2]).reshape(C, NCH, G)

    full = lambda shp: pl.BlockSpec(shp, lambda *_: tuple(0 for _ in shp))
    out = pl.pallas_call(
        _raster_kernel,
        grid_spec=pltpu.PrefetchScalarGridSpec(
            num_scalar_prefetch=1,
            grid=(grid,),
            in_specs=[full((NCH, G))] * 5 + [full((C, NCH, G))],
            out_specs=pl.BlockSpec((C, ROWS, W), lambda i, b: (0, i, 0)),
        ),
        out_shape=jax.ShapeDtypeStruct((C, H, W), jnp.float32),
    )(bounds, q(cx_s), q(cy_s), q(a_s), q(b_s), q(c_s), vt)

    return out.reshape(1, C, H, W)
